# baseline (speedup 1.0000x reference)
import functools

import jax
import jax.numpy as jnp
from jax import lax
from jax.experimental import pallas as pl
from jax.experimental.pallas import tpu as pltpu
from jax.experimental.pallas import tpu_sc as plsc

_K = 1024
_D = 64
_N = 32768
_CH = 4096
_COMMIT = 0.25

_NC = 2
_NS = 16
_NW = _NC * _NS
_RPW = _N // _NW
_HALF = _RPW // 2


def _argmin_body(x_ref, wt_ref, wt2_ref, idx_ref, loss_ref, lsum):
    b = pl.program_id(0)
    j = pl.program_id(1)
    nb = pl.num_programs(0)
    nj = pl.num_programs(1)

    x = x_ref[...]
    wt = wt_ref[...]
    wt2 = wt2_ref[...]

    s2 = lax.dot_general(x, wt2, (((1,), (0,)), ((), ())),
                         preferred_element_type=jnp.float32)
    xn = jnp.sum(x * x, axis=1, keepdims=True)
    wn = jnp.sum(wt * wt, axis=0, keepdims=True)
    d = (xn + wn) - s2

    m = jnp.min(d, axis=1, keepdims=True)
    iot = lax.broadcasted_iota(jnp.int32, d.shape, 1)
    idx = jnp.min(jnp.where(d == m, iot, _K), axis=1, keepdims=True)
    idx_ref[0] = jnp.transpose(idx)

    @pl.when(jnp.logical_and(b == 0, j == 0))
    def _init():
        lsum[...] = jnp.zeros_like(lsum)

    lsum[...] += jnp.sum(m, keepdims=True).reshape(1, 1)

    @pl.when(jnp.logical_and(b == nb - 1, j == nj - 1))
    def _finalize():
        mse = lsum[...] / float(_N * _D)
        loss_ref[...] = (1.0 + _COMMIT) * mse


def _sc_gather_body(wt_hbm, idx_hbm, q_hbm, hist_hbm,
                    wt_v, buf, idx_v, hist2, histv, sem):
    wid = lax.axis_index("s") * _NC + lax.axis_index("c")
    base = wid * _RPW
    bb = wid // 8
    j0 = (wid % 8) * _RPW

    pltpu.sync_copy(wt_hbm, wt_v)
    pltpu.sync_copy(idx_hbm.at[pl.ds(base, _RPW)], idx_v)

    zero16 = jnp.zeros((16,), jnp.float32)
    ones16 = jnp.ones((16,), jnp.float32)
    lane16 = lax.iota(jnp.int32, 16)

    def _zero(i, _):
        hist2[pl.ds(i * 16, 16)] = zero16
        return _
    lax.fori_loop(0, (16 * _K) // 16, _zero, None)

    for half in range(2):
        def _gather(g, _, _half=half):
            off = _half * _HALF + g * 16
            idx16 = idx_v[pl.ds(off, 16)]
            plsc.addupdate_scatter(hist2, [lane16 * _K + idx16], ones16)
            for c in range(_D):
                c16 = jnp.full((16,), c, jnp.int32)
                buf[c, pl.ds(g * 16, 16)] = plsc.load_gather(wt_v, [c16, idx16])
            return _
        lax.fori_loop(0, _HALF // 16, _gather, None)
        pltpu.sync_copy(buf, q_hbm.at[bb, :, pl.ds(j0 + half * _HALF, _HALF)])

    def _reduce(v, _):
        acc = zero16
        for l in range(16):
            acc = acc + hist2[pl.ds(l * _K + v * 16, 16)]
        histv[pl.ds(v * 16, 16)] = acc
        return _
    lax.fori_loop(0, _K // 16, _reduce, None)
    pltpu.sync_copy(histv, hist_hbm.at[wid])


def _perp_body(hist_ref, perp_ref):
    hist = jnp.sum(hist_ref[...], axis=0, keepdims=True)
    p = hist / float(_N)
    ent = jnp.sum(p * jnp.log(p + 1e-10), axis=1, keepdims=True)
    perp_ref[...] = jnp.exp(-ent)


def kernel(inputs, embedding_weight):
    b, c, t, h, w = inputs.shape
    flat = jnp.transpose(inputs, (0, 2, 3, 4, 1)).reshape(-1, c)
    wt = embedding_weight.T
    wt2 = wt + wt

    nj = (t * h * w) // _CH
    idx, loss = pl.pallas_call(
        _argmin_body,
        grid=(b, nj),
        in_specs=[
            pl.BlockSpec((_CH, _D), lambda i, j, _nj=nj: (i * _nj + j, 0)),
            pl.BlockSpec((_D, _K), lambda i, j: (0, 0)),
            pl.BlockSpec((_D, _K), lambda i, j: (0, 0)),
        ],
        out_specs=[
            pl.BlockSpec((1, 1, _CH), lambda i, j, _nj=nj: (i * _nj + j, 0, 0)),
            pl.BlockSpec((1, 1), lambda i, j: (0, 0)),
        ],
        out_shape=[
            jax.ShapeDtypeStruct((_N // _CH, 1, _CH), jnp.int32),
            jax.ShapeDtypeStruct((1, 1), jnp.float32),
        ],
        scratch_shapes=[
            pltpu.VMEM((1, 1), jnp.float32),
        ],
    )(flat, wt, wt2)

    mesh = plsc.VectorSubcoreMesh(core_axis_name="c", subcore_axis_name="s")
    sc_gather = functools.partial(
        pl.kernel,
        out_type=[
            jax.ShapeDtypeStruct((b, c, t * h * w), jnp.float32),
            jax.ShapeDtypeStruct((_NW, _K), jnp.float32),
        ],
        mesh=mesh,
        scratch_types=[
            pltpu.VMEM((_D, _K), jnp.float32),
            pltpu.VMEM((_D, _HALF), jnp.float32),
            pltpu.VMEM((_RPW,), jnp.int32),
            pltpu.VMEM((16 * _K,), jnp.float32),
            pltpu.VMEM((_K,), jnp.float32),
            pltpu.SemaphoreType.DMA,
        ],
        compiler_params=pltpu.CompilerParams(needs_layout_passes=False),
    )(_sc_gather_body)
    qt, hist32 = sc_gather(wt, idx.reshape(-1))

    perp = pl.pallas_call(
        _perp_body,
        grid=(1,),
        in_specs=[pl.BlockSpec((_NW, _K), lambda i: (0, 0))],
        out_specs=[pl.BlockSpec((1, 1), lambda i: (0, 0))],
        out_shape=[jax.ShapeDtypeStruct((1, 1), jnp.float32)],
    )(hist32)[0]

    quantized = qt.reshape(b, c, t, h, w)
    return quantized, loss[0, 0], perp[0, 0]


# speedup vs baseline: 1.5532x; 1.5532x over previous
import jax
import jax.numpy as jnp
from jax import lax
from jax.experimental import pallas as pl
from jax.experimental.pallas import tpu as pltpu

_K = 1024
_D = 64
_N = 32768
_CH = 4096
_COMMIT = 0.25


def _vq_body(x_ref, wt_ref, wt2_ref, q_ref, loss_ref, perp_ref, lsum, hist):
    b = pl.program_id(0)
    j = pl.program_id(1)
    nb = pl.num_programs(0)
    nj = pl.num_programs(1)

    x = x_ref[...]
    wt = wt_ref[...]
    wt2 = wt2_ref[...]

    s2 = lax.dot_general(x, wt2, (((1,), (0,)), ((), ())),
                         preferred_element_type=jnp.float32)
    xn = jnp.sum(x * x, axis=1, keepdims=True)
    wn = jnp.sum(wt * wt, axis=0, keepdims=True)
    d = (xn + wn) - s2

    m = jnp.min(d, axis=1, keepdims=True)
    iot = lax.broadcasted_iota(jnp.int32, d.shape, 1).astype(jnp.float32)
    idx = jnp.min(jnp.where(d == m, iot, float(_K)), axis=1, keepdims=True)
    onehot = (iot == idx).astype(jnp.float32)

    qt = lax.dot_general(wt, onehot, (((1,), (1,)), ((), ())),
                         preferred_element_type=jnp.float32)
    q_ref[0] = qt

    @pl.when(jnp.logical_and(b == 0, j == 0))
    def _init():
        lsum[...] = jnp.zeros_like(lsum)
        hist[...] = jnp.zeros_like(hist)

    ones_row = jnp.ones((1, _CH), jnp.float32)
    lsum[...] += jnp.sum(m, keepdims=True).reshape(1, 1)
    hist[...] += lax.dot_general(ones_row, onehot, (((1,), (0,)), ((), ())),
                                 preferred_element_type=jnp.float32)

    @pl.when(jnp.logical_and(b == nb - 1, j == nj - 1))
    def _finalize():
        mse = lsum[...] / float(_N * _D)
        loss_ref[...] = (1.0 + _COMMIT) * mse
        p = hist[...] / float(_N)
        ent = jnp.sum(p * jnp.log(p + 1e-10), axis=1, keepdims=True)
        perp_ref[...] = jnp.exp(-ent)


def kernel(inputs, embedding_weight):
    b, c, t, h, w = inputs.shape
    flat = jnp.transpose(inputs, (0, 2, 3, 4, 1)).reshape(-1, c)
    wt = embedding_weight.T
    wt2 = wt + wt

    nj = (t * h * w) // _CH
    qt, loss, perp = pl.pallas_call(
        _vq_body,
        grid=(b, nj),
        in_specs=[
            pl.BlockSpec((_CH, _D), lambda i, j, _nj=nj: (i * _nj + j, 0)),
            pl.BlockSpec((_D, _K), lambda i, j: (0, 0)),
            pl.BlockSpec((_D, _K), lambda i, j: (0, 0)),
        ],
        out_specs=[
            pl.BlockSpec((1, _D, _CH), lambda i, j: (i, 0, j)),
            pl.BlockSpec((1, 1), lambda i, j: (0, 0)),
            pl.BlockSpec((1, 1), lambda i, j: (0, 0)),
        ],
        out_shape=[
            jax.ShapeDtypeStruct((b, _D, t * h * w), jnp.float32),
            jax.ShapeDtypeStruct((1, 1), jnp.float32),
            jax.ShapeDtypeStruct((1, 1), jnp.float32),
        ],
        scratch_shapes=[
            pltpu.VMEM((1, 1), jnp.float32),
            pltpu.VMEM((1, _K), jnp.float32),
        ],
    )(flat, wt, wt2)

    quantized = qt.reshape(b, c, t, h, w)
    return quantized, loss[0, 0], perp[0, 0]
